# baseline (speedup 1.0000x reference)
import jax
import jax.numpy as jnp
from jax.experimental import pallas as pl
from jax.experimental.pallas import tpu as pltpu

E = 8
TOPK = 2
DIM = 768
HID = 3072

BT = 256
NP = 8192 + E * BT
NB = NP // BT
BTA = 256
BTL = 1024
BTC = 512

_DNT = (((1,), (1,)), ((), ()))


def _dot_t(a, b):
    return jax.lax.dot_general(a, b, _DNT, preferred_element_type=jnp.float32)


def _router_kernel(x_ref, wr_ref, log_ref, oh1_ref, oh2_ref, w_ref):
    x = x_ref[...].astype(jnp.bfloat16)
    logits = _dot_t(x, wr_ref[...])
    log_ref[...] = logits
    ii = jax.lax.broadcasted_iota(jnp.int32, logits.shape, 1)
    l1 = jnp.max(logits, axis=1, keepdims=True)
    i1 = jnp.min(jnp.where(logits == l1, ii, E), axis=1, keepdims=True)
    oh1 = ii == i1
    masked = jnp.where(oh1, -jnp.inf, logits)
    l2 = jnp.max(masked, axis=1, keepdims=True)
    i2 = jnp.min(jnp.where(masked == l2, ii, E), axis=1, keepdims=True)
    oh2 = ii == i2
    oh1_ref[...] = oh1.astype(jnp.int32)
    oh2_ref[...] = oh2.astype(jnp.int32)
    wa = 1.0 / (1.0 + jnp.exp(l2 - l1))
    w_ref[...] = jnp.concatenate([wa, 1.0 - wa], axis=1)


def _shared_kernel(x_ref, w1_ref, w2_ref, w3_ref, sh_ref):
    x = x_ref[...].astype(jnp.bfloat16)
    h1 = _dot_t(x, w1_ref[...])
    h2 = _dot_t(x, w2_ref[...])
    h = (jax.nn.silu(h1) * h2).astype(jnp.bfloat16)
    sh_ref[...] = _dot_t(h, w3_ref[...])


def _grouped_ffn_kernel(be_ref, x_ref, rw_ref, w1_ref, w2_ref, w3_ref, y_ref):
    x = x_ref[...].astype(jnp.bfloat16)
    h1 = _dot_t(x, w1_ref[0])
    h2 = _dot_t(x, w2_ref[0])
    h = (jax.nn.silu(h1) * h2).astype(jnp.bfloat16)
    y = _dot_t(h, w3_ref[0])
    y_ref[...] = y * rw_ref[...]


def _combine_kernel(sh_ref, g_ref, o_ref):
    o_ref[...] = sh_ref[...] + g_ref[:, 0, :] + g_ref[:, 1, :]


def kernel(x, W1, W2, W3, SW1, SW2, SW3, Wr):
    batch, seq, dim = x.shape
    x_flat = x.reshape(-1, dim)
    T = x_flat.shape[0]
    P = T * TOPK

    sw1 = SW1[0].astype(jnp.bfloat16)
    sw2 = SW2[0].astype(jnp.bfloat16)
    sw3 = SW3[0].astype(jnp.bfloat16)
    w1b = W1.astype(jnp.bfloat16)
    w2b = W2.astype(jnp.bfloat16)
    w3b = W3.astype(jnp.bfloat16)
    wrb = Wr.astype(jnp.bfloat16)

    logits, oh1, oh2, top_w = pl.pallas_call(
        _router_kernel,
        grid=(T // BTL,),
        in_specs=[
            pl.BlockSpec((BTL, DIM), lambda i: (i, 0)),
            pl.BlockSpec((E, DIM), lambda i: (0, 0)),
        ],
        out_specs=[
            pl.BlockSpec((BTL, E), lambda i: (i, 0)),
            pl.BlockSpec((BTL, E), lambda i: (i, 0)),
            pl.BlockSpec((BTL, E), lambda i: (i, 0)),
            pl.BlockSpec((BTL, TOPK), lambda i: (i, 0)),
        ],
        out_shape=[
            jax.ShapeDtypeStruct((T, E), jnp.float32),
            jax.ShapeDtypeStruct((T, E), jnp.int32),
            jax.ShapeDtypeStruct((T, E), jnp.int32),
            jax.ShapeDtypeStruct((T, TOPK), jnp.float32),
        ],
    )(x_flat, wrb)

    oh = oh1 + oh2
    c_incl = oh * 2
    c_excl = c_incl - oh
    counts = c_incl[-1]
    pc = ((counts + BT - 1) // BT) * BT
    starts = (jnp.concatenate([jnp.zeros(1, pc.dtype), jnp.cumsum(pc)[:-1]])
              .astype(jnp.int32))
    base = c_excl + starts[None, :]
    pos1 = jnp.sum(oh1 * base, axis=1)
    pos2 = jnp.sum(oh2 * base, axis=1)
    pos_flat = jnp.stack([pos1, pos2], axis=1).reshape(P)
    token_pair = jnp.repeat(jnp.arange(T, dtype=jnp.int32), TOPK)
    row_token = jnp.zeros(NP, jnp.int32).at[pos_flat].set(token_pair)
    row_w = jnp.zeros((NP, 1), jnp.float32).at[pos_flat, 0].set(
        top_w.reshape(P))
    block_expert = (jnp.searchsorted(starts, jnp.arange(NB) * BT, side='right')
                    .astype(jnp.int32) - 1)

    x_sorted = jnp.take(x_flat, row_token, axis=0)

    shared_out = pl.pallas_call(
        _shared_kernel,
        grid=(T // BTA,),
        in_specs=[
            pl.BlockSpec((BTA, DIM), lambda i: (i, 0)),
            pl.BlockSpec((HID, DIM), lambda i: (0, 0)),
            pl.BlockSpec((HID, DIM), lambda i: (0, 0)),
            pl.BlockSpec((DIM, HID), lambda i: (0, 0)),
        ],
        out_specs=pl.BlockSpec((BTA, DIM), lambda i: (i, 0)),
        out_shape=jax.ShapeDtypeStruct((T, DIM), jnp.float32),
    )(x_flat, sw1, sw2, sw3)

    grid_spec = pltpu.PrefetchScalarGridSpec(
        num_scalar_prefetch=1,
        grid=(NB,),
        in_specs=[
            pl.BlockSpec((BT, DIM), lambda b, be: (b, 0)),
            pl.BlockSpec((BT, 1), lambda b, be: (b, 0)),
            pl.BlockSpec((1, HID, DIM), lambda b, be: (be[b], 0, 0)),
            pl.BlockSpec((1, HID, DIM), lambda b, be: (be[b], 0, 0)),
            pl.BlockSpec((1, DIM, HID), lambda b, be: (be[b], 0, 0)),
        ],
        out_specs=pl.BlockSpec((BT, DIM), lambda b, be: (b, 0)),
    )
    y_sorted = pl.pallas_call(
        _grouped_ffn_kernel,
        grid_spec=grid_spec,
        out_shape=jax.ShapeDtypeStruct((NP, DIM), jnp.float32),
    )(block_expert, x_sorted, row_w, w1b, w2b, w3b)

    g = jnp.take(y_sorted, pos_flat, axis=0).reshape(T, TOPK, DIM)
    out = pl.pallas_call(
        _combine_kernel,
        grid=(T // BTC,),
        in_specs=[
            pl.BlockSpec((BTC, DIM), lambda i: (i, 0)),
            pl.BlockSpec((BTC, TOPK, DIM), lambda i: (i, 0, 0)),
        ],
        out_specs=pl.BlockSpec((BTC, DIM), lambda i: (i, 0)),
        out_shape=jax.ShapeDtypeStruct((T, DIM), jnp.float32),
    )(shared_out, g)

    return (out.reshape(batch, seq, dim), logits)


# speedup vs baseline: 1.2554x; 1.2554x over previous
import jax
import jax.numpy as jnp
from jax.experimental import pallas as pl
from jax.experimental.pallas import tpu as pltpu

E = 8
TOPK = 2
DIM = 768
HID = 3072

BT = 256
NP = 8192 + E * BT
NB = NP // BT
BTA = 256
BTL = 1024
BTC = 512

_DNT = (((1,), (1,)), ((), ()))


def _dot_t(a, b):
    return jax.lax.dot_general(a, b, _DNT, preferred_element_type=jnp.float32)


def _router_kernel(x_ref, wr_ref, log_ref, oh1_ref, oh2_ref, w_ref):
    x = x_ref[...].astype(jnp.bfloat16)
    logits = _dot_t(x, wr_ref[...])
    log_ref[...] = logits
    ii = jax.lax.broadcasted_iota(jnp.int32, logits.shape, 1)
    l1 = jnp.max(logits, axis=1, keepdims=True)
    i1 = jnp.min(jnp.where(logits == l1, ii, E), axis=1, keepdims=True)
    oh1 = ii == i1
    masked = jnp.where(oh1, -jnp.inf, logits)
    l2 = jnp.max(masked, axis=1, keepdims=True)
    i2 = jnp.min(jnp.where(masked == l2, ii, E), axis=1, keepdims=True)
    oh2 = ii == i2
    oh1_ref[...] = oh1.astype(jnp.int32)
    oh2_ref[...] = oh2.astype(jnp.int32)
    wa = 1.0 / (1.0 + jnp.exp(l2 - l1))
    w_ref[...] = jnp.concatenate([wa, 1.0 - wa], axis=1)


def _shared_kernel(x_ref, w1_ref, w2_ref, w3_ref, sh_ref):
    x = x_ref[...].astype(jnp.bfloat16)
    h1 = _dot_t(x, w1_ref[...])
    h2 = _dot_t(x, w2_ref[...])
    h = (jax.nn.silu(h1) * h2).astype(jnp.bfloat16)
    sh_ref[...] = _dot_t(h, w3_ref[...])


def _grouped_ffn_kernel(be_ref, x_ref, rw_ref, w1_ref, w2_ref, w3_ref, y_ref):
    x = x_ref[...].astype(jnp.bfloat16)
    h1 = _dot_t(x, w1_ref[0])
    h2 = _dot_t(x, w2_ref[0])
    h = (jax.nn.silu(h1) * h2).astype(jnp.bfloat16)
    y = _dot_t(h, w3_ref[0])
    y_ref[...] = y * rw_ref[...]


def _combine_kernel(sh_ref, g_ref, o_ref):
    o_ref[...] = sh_ref[...] + g_ref[:, 0, :] + g_ref[:, 1, :]


def kernel(x, W1, W2, W3, SW1, SW2, SW3, Wr):
    batch, seq, dim = x.shape
    x_flat = x.reshape(-1, dim)
    T = x_flat.shape[0]
    P = T * TOPK

    sw1 = SW1[0].astype(jnp.bfloat16)
    sw2 = SW2[0].astype(jnp.bfloat16)
    sw3 = SW3[0].astype(jnp.bfloat16)
    w1b = W1.astype(jnp.bfloat16)
    w2b = W2.astype(jnp.bfloat16)
    w3b = W3.astype(jnp.bfloat16)
    wrb = Wr.astype(jnp.bfloat16)

    logits, oh1, oh2, top_w = pl.pallas_call(
        _router_kernel,
        grid=(T // BTL,),
        in_specs=[
            pl.BlockSpec((BTL, DIM), lambda i: (i, 0)),
            pl.BlockSpec((E, DIM), lambda i: (0, 0)),
        ],
        out_specs=[
            pl.BlockSpec((BTL, E), lambda i: (i, 0)),
            pl.BlockSpec((BTL, E), lambda i: (i, 0)),
            pl.BlockSpec((BTL, E), lambda i: (i, 0)),
            pl.BlockSpec((BTL, TOPK), lambda i: (i, 0)),
        ],
        out_shape=[
            jax.ShapeDtypeStruct((T, E), jnp.float32),
            jax.ShapeDtypeStruct((T, E), jnp.int32),
            jax.ShapeDtypeStruct((T, E), jnp.int32),
            jax.ShapeDtypeStruct((T, TOPK), jnp.float32),
        ],
    )(x_flat, wrb)

    oh = oh1 + oh2
    c_incl = jnp.cumsum(oh, axis=0)
    c_excl = c_incl - oh
    counts = c_incl[-1]
    pc = ((counts + BT - 1) // BT) * BT
    starts = (jnp.concatenate([jnp.zeros(1, pc.dtype), jnp.cumsum(pc)[:-1]])
              .astype(jnp.int32))
    base = c_excl + starts[None, :]
    pos1 = jnp.sum(oh1 * base, axis=1)
    pos2 = jnp.sum(oh2 * base, axis=1)
    pos_flat = jnp.stack([pos1, pos2], axis=1).reshape(P)
    token_pair = jnp.repeat(jnp.arange(T, dtype=jnp.int32), TOPK)
    row_token = jnp.zeros(NP, jnp.int32).at[pos_flat].set(token_pair)
    row_w = jnp.zeros((NP, 1), jnp.float32).at[pos_flat, 0].set(
        top_w.reshape(P))
    block_expert = (jnp.searchsorted(starts, jnp.arange(NB) * BT, side='right')
                    .astype(jnp.int32) - 1)

    x_sorted = jnp.take(x_flat, row_token, axis=0)

    shared_out = pl.pallas_call(
        _shared_kernel,
        grid=(T // BTA,),
        in_specs=[
            pl.BlockSpec((BTA, DIM), lambda i: (i, 0)),
            pl.BlockSpec((HID, DIM), lambda i: (0, 0)),
            pl.BlockSpec((HID, DIM), lambda i: (0, 0)),
            pl.BlockSpec((DIM, HID), lambda i: (0, 0)),
        ],
        out_specs=pl.BlockSpec((BTA, DIM), lambda i: (i, 0)),
        out_shape=jax.ShapeDtypeStruct((T, DIM), jnp.float32),
    )(x_flat, sw1, sw2, sw3)

    grid_spec = pltpu.PrefetchScalarGridSpec(
        num_scalar_prefetch=1,
        grid=(NB,),
        in_specs=[
            pl.BlockSpec((BT, DIM), lambda b, be: (b, 0)),
            pl.BlockSpec((BT, 1), lambda b, be: (b, 0)),
            pl.BlockSpec((1, HID, DIM), lambda b, be: (be[b], 0, 0)),
            pl.BlockSpec((1, HID, DIM), lambda b, be: (be[b], 0, 0)),
            pl.BlockSpec((1, DIM, HID), lambda b, be: (be[b], 0, 0)),
        ],
        out_specs=pl.BlockSpec((BT, DIM), lambda b, be: (b, 0)),
    )
    y_sorted = pl.pallas_call(
        _grouped_ffn_kernel,
        grid_spec=grid_spec,
        out_shape=jax.ShapeDtypeStruct((NP, DIM), jnp.float32),
    )(block_expert, x_sorted, row_w, w1b, w2b, w3b)

    g = jnp.take(y_sorted, pos_flat, axis=0).reshape(T, TOPK, DIM)
    out = pl.pallas_call(
        _combine_kernel,
        grid=(T // BTC,),
        in_specs=[
            pl.BlockSpec((BTC, DIM), lambda i: (i, 0)),
            pl.BlockSpec((BTC, TOPK, DIM), lambda i: (i, 0, 0)),
        ],
        out_specs=pl.BlockSpec((BTC, DIM), lambda i: (i, 0)),
        out_shape=jax.ShapeDtypeStruct((T, DIM), jnp.float32),
    )(shared_out, g)

    return (out.reshape(batch, seq, dim), logits)


# speedup vs baseline: 1.5463x; 1.2317x over previous
import jax
import jax.numpy as jnp
from jax.experimental import pallas as pl
from jax.experimental.pallas import tpu as pltpu

E = 8
TOPK = 2
DIM = 768
HID = 3072

BT = 256
NP = 8192 + E * BT
NB = NP // BT
BTA = 256
BTL = 1024
BTC = 512

_DNT = (((1,), (1,)), ((), ()))


def _dot_t(a, b):
    return jax.lax.dot_general(a, b, _DNT, preferred_element_type=jnp.float32)


def _router_kernel(x_ref, wr_ref, log_ref, oh1_ref, oh2_ref, w_ref):
    x = x_ref[...].astype(jnp.bfloat16)
    logits = _dot_t(x, wr_ref[...])
    log_ref[...] = logits
    ii = jax.lax.broadcasted_iota(jnp.int32, logits.shape, 1)
    l1 = jnp.max(logits, axis=1, keepdims=True)
    i1 = jnp.min(jnp.where(logits == l1, ii, E), axis=1, keepdims=True)
    oh1 = ii == i1
    masked = jnp.where(oh1, -jnp.inf, logits)
    l2 = jnp.max(masked, axis=1, keepdims=True)
    i2 = jnp.min(jnp.where(masked == l2, ii, E), axis=1, keepdims=True)
    oh2 = ii == i2
    oh1_ref[...] = oh1.astype(jnp.int32)
    oh2_ref[...] = oh2.astype(jnp.int32)
    wa = 1.0 / (1.0 + jnp.exp(l2 - l1))
    w_ref[...] = jnp.concatenate([wa, 1.0 - wa], axis=1)


def _shared_kernel(x_ref, w1_ref, w2_ref, w3_ref, sh_ref):
    x = x_ref[...].astype(jnp.bfloat16)
    h1 = _dot_t(x, w1_ref[...])
    h2 = _dot_t(x, w2_ref[...])
    h = (jax.nn.silu(h1) * h2).astype(jnp.bfloat16)
    sh_ref[...] = _dot_t(h, w3_ref[...])


def _grouped_ffn_kernel(be_ref, x_ref, w1_ref, w2_ref, w3_ref, y_ref):
    x = x_ref[...].astype(jnp.bfloat16)
    h1 = _dot_t(x, w1_ref[0])
    h2 = _dot_t(x, w2_ref[0])
    h = (jax.nn.silu(h1) * h2).astype(jnp.bfloat16)
    y_ref[...] = _dot_t(h, w3_ref[0])


def _combine_kernel(sh_ref, g1_ref, g2_ref, w_ref, o_ref):
    w = w_ref[...]
    o_ref[...] = (sh_ref[...] + w[:, 0:1] * g1_ref[...]
                  + w[:, 1:2] * g2_ref[...])


def kernel(x, W1, W2, W3, SW1, SW2, SW3, Wr):
    batch, seq, dim = x.shape
    x_flat = x.reshape(-1, dim)
    T = x_flat.shape[0]
    P = T * TOPK

    sw1 = SW1[0].astype(jnp.bfloat16)
    sw2 = SW2[0].astype(jnp.bfloat16)
    sw3 = SW3[0].astype(jnp.bfloat16)
    w1b = W1.astype(jnp.bfloat16)
    w2b = W2.astype(jnp.bfloat16)
    w3b = W3.astype(jnp.bfloat16)
    wrb = Wr.astype(jnp.bfloat16)

    logits, oh1, oh2, top_w = pl.pallas_call(
        _router_kernel,
        grid=(T // BTL,),
        in_specs=[
            pl.BlockSpec((BTL, DIM), lambda i: (i, 0)),
            pl.BlockSpec((E, DIM), lambda i: (0, 0)),
        ],
        out_specs=[
            pl.BlockSpec((BTL, E), lambda i: (i, 0)),
            pl.BlockSpec((BTL, E), lambda i: (i, 0)),
            pl.BlockSpec((BTL, E), lambda i: (i, 0)),
            pl.BlockSpec((BTL, TOPK), lambda i: (i, 0)),
        ],
        out_shape=[
            jax.ShapeDtypeStruct((T, E), jnp.float32),
            jax.ShapeDtypeStruct((T, E), jnp.int32),
            jax.ShapeDtypeStruct((T, E), jnp.int32),
            jax.ShapeDtypeStruct((T, TOPK), jnp.float32),
        ],
    )(x_flat, wrb)

    oh = oh1 + oh2
    c_incl = jnp.cumsum(oh, axis=0)
    c_excl = c_incl - oh
    counts = c_incl[-1]
    pc = ((counts + BT - 1) // BT) * BT
    starts = (jnp.concatenate([jnp.zeros(1, pc.dtype), jnp.cumsum(pc)[:-1]])
              .astype(jnp.int32))
    base = c_excl + starts[None, :]
    pos1 = jnp.sum(oh1 * base, axis=1).astype(jnp.int32)
    pos2 = jnp.sum(oh2 * base, axis=1).astype(jnp.int32)
    tok = jnp.arange(T, dtype=jnp.int32)
    row_token = (jnp.zeros(NP, jnp.int32)
                 .at[jnp.concatenate([pos1, pos2])]
                 .set(jnp.concatenate([tok, tok])))
    block_expert = (jnp.sum(starts[None, :]
                            <= (jnp.arange(NB, dtype=jnp.int32) * BT)[:, None],
                            axis=1).astype(jnp.int32) - 1)

    x_sorted = jnp.take(x_flat, row_token, axis=0)

    shared_out = pl.pallas_call(
        _shared_kernel,
        grid=(T // BTA,),
        in_specs=[
            pl.BlockSpec((BTA, DIM), lambda i: (i, 0)),
            pl.BlockSpec((HID, DIM), lambda i: (0, 0)),
            pl.BlockSpec((HID, DIM), lambda i: (0, 0)),
            pl.BlockSpec((DIM, HID), lambda i: (0, 0)),
        ],
        out_specs=pl.BlockSpec((BTA, DIM), lambda i: (i, 0)),
        out_shape=jax.ShapeDtypeStruct((T, DIM), jnp.float32),
    )(x_flat, sw1, sw2, sw3)

    grid_spec = pltpu.PrefetchScalarGridSpec(
        num_scalar_prefetch=1,
        grid=(NB,),
        in_specs=[
            pl.BlockSpec((BT, DIM), lambda b, be: (b, 0)),
            pl.BlockSpec((1, HID, DIM), lambda b, be: (be[b], 0, 0)),
            pl.BlockSpec((1, HID, DIM), lambda b, be: (be[b], 0, 0)),
            pl.BlockSpec((1, DIM, HID), lambda b, be: (be[b], 0, 0)),
        ],
        out_specs=pl.BlockSpec((BT, DIM), lambda b, be: (b, 0)),
    )
    y_sorted = pl.pallas_call(
        _grouped_ffn_kernel,
        grid_spec=grid_spec,
        out_shape=jax.ShapeDtypeStruct((NP, DIM), jnp.float32),
    )(block_expert, x_sorted, w1b, w2b, w3b)

    g1 = jnp.take(y_sorted, pos1, axis=0)
    g2 = jnp.take(y_sorted, pos2, axis=0)
    out = pl.pallas_call(
        _combine_kernel,
        grid=(T // BTC,),
        in_specs=[
            pl.BlockSpec((BTC, DIM), lambda i: (i, 0)),
            pl.BlockSpec((BTC, DIM), lambda i: (i, 0)),
            pl.BlockSpec((BTC, DIM), lambda i: (i, 0)),
            pl.BlockSpec((BTC, TOPK), lambda i: (i, 0)),
        ],
        out_specs=pl.BlockSpec((BTC, DIM), lambda i: (i, 0)),
        out_shape=jax.ShapeDtypeStruct((T, DIM), jnp.float32),
    )(shared_out, g1, g2, top_w)

    return (out.reshape(batch, seq, dim), logits)


# speedup vs baseline: 1.5584x; 1.0078x over previous
import jax
import jax.numpy as jnp
from jax.experimental import pallas as pl
from jax.experimental.pallas import tpu as pltpu

E = 8
TOPK = 2
DIM = 768
HID = 3072

BT = 256
NP = 8192 + E * BT
NB = NP // BT
BTA = 256
BTL = 1024
BTC = 512

_DNT = (((1,), (1,)), ((), ()))


def _dot_t(a, b):
    return jax.lax.dot_general(a, b, _DNT, preferred_element_type=jnp.float32)


def _router_kernel(x_ref, wr_ref, log_ref, oh1_ref, oh2_ref, w_ref):
    x = x_ref[...].astype(jnp.bfloat16)
    logits = _dot_t(x, wr_ref[...])
    log_ref[...] = logits
    ii = jax.lax.broadcasted_iota(jnp.int32, logits.shape, 1)
    l1 = jnp.max(logits, axis=1, keepdims=True)
    i1 = jnp.min(jnp.where(logits == l1, ii, E), axis=1, keepdims=True)
    oh1 = ii == i1
    masked = jnp.where(oh1, -jnp.inf, logits)
    l2 = jnp.max(masked, axis=1, keepdims=True)
    i2 = jnp.min(jnp.where(masked == l2, ii, E), axis=1, keepdims=True)
    oh2 = ii == i2
    oh1_ref[...] = oh1.astype(jnp.float32)
    oh2_ref[...] = oh2.astype(jnp.float32)
    wa = 1.0 / (1.0 + jnp.exp(l2 - l1))
    w_ref[...] = jnp.concatenate([wa, 1.0 - wa], axis=1)


def _shared_kernel(x_ref, w1_ref, w2_ref, w3_ref, sh_ref):
    x = x_ref[...].astype(jnp.bfloat16)
    h1 = _dot_t(x, w1_ref[...])
    h2 = _dot_t(x, w2_ref[...])
    h = (jax.nn.silu(h1) * h2).astype(jnp.bfloat16)
    sh_ref[...] = _dot_t(h, w3_ref[...])


def _grouped_ffn_kernel(be_ref, x_ref, w1_ref, w2_ref, w3_ref, y_ref):
    x = x_ref[...].astype(jnp.bfloat16)
    h1 = _dot_t(x, w1_ref[0])
    h2 = _dot_t(x, w2_ref[0])
    h = (jax.nn.silu(h1) * h2).astype(jnp.bfloat16)
    y_ref[...] = _dot_t(h, w3_ref[0])


def _combine_kernel(sh_ref, g1_ref, g2_ref, w_ref, o_ref):
    w = w_ref[...]
    o_ref[...] = (sh_ref[...] + w[:, 0:1] * g1_ref[...]
                  + w[:, 1:2] * g2_ref[...])


def kernel(x, W1, W2, W3, SW1, SW2, SW3, Wr):
    batch, seq, dim = x.shape
    x_flat = x.reshape(-1, dim)
    T = x_flat.shape[0]
    P = T * TOPK

    sw1 = SW1[0].astype(jnp.bfloat16)
    sw2 = SW2[0].astype(jnp.bfloat16)
    sw3 = SW3[0].astype(jnp.bfloat16)
    w1b = W1.astype(jnp.bfloat16)
    w2b = W2.astype(jnp.bfloat16)
    w3b = W3.astype(jnp.bfloat16)
    wrb = Wr.astype(jnp.bfloat16)

    logits, oh1, oh2, top_w = pl.pallas_call(
        _router_kernel,
        grid=(T // BTL,),
        in_specs=[
            pl.BlockSpec((BTL, DIM), lambda i: (i, 0)),
            pl.BlockSpec((E, DIM), lambda i: (0, 0)),
        ],
        out_specs=[
            pl.BlockSpec((BTL, E), lambda i: (i, 0)),
            pl.BlockSpec((BTL, E), lambda i: (i, 0)),
            pl.BlockSpec((BTL, E), lambda i: (i, 0)),
            pl.BlockSpec((BTL, TOPK), lambda i: (i, 0)),
        ],
        out_shape=[
            jax.ShapeDtypeStruct((T, E), jnp.float32),
            jax.ShapeDtypeStruct((T, E), jnp.float32),
            jax.ShapeDtypeStruct((T, E), jnp.float32),
            jax.ShapeDtypeStruct((T, TOPK), jnp.float32),
        ],
    )(x_flat, wrb)

    oh = oh1 + oh2
    c_incl = jnp.cumsum(oh, axis=0)
    c_excl = c_incl - oh
    counts = c_incl[-1].astype(jnp.int32)
    pc = ((counts + BT - 1) // BT) * BT
    starts = (jnp.concatenate([jnp.zeros(1, pc.dtype), jnp.cumsum(pc)[:-1]])
              .astype(jnp.int32))
    base = c_excl + starts[None, :].astype(jnp.float32)
    pos1 = jnp.sum(oh1 * base, axis=1).astype(jnp.int32)
    pos2 = jnp.sum(oh2 * base, axis=1).astype(jnp.int32)
    tok = jnp.arange(T, dtype=jnp.int32)
    row_token = (jnp.zeros(NP, jnp.int32)
                 .at[jnp.concatenate([pos1, pos2])]
                 .set(jnp.concatenate([tok, tok])))
    block_expert = (jnp.sum(starts[None, :]
                            <= (jnp.arange(NB, dtype=jnp.int32) * BT)[:, None],
                            axis=1).astype(jnp.int32) - 1)

    x_sorted = jnp.take(x_flat, row_token, axis=0)

    shared_out = pl.pallas_call(
        _shared_kernel,
        grid=(T // BTA,),
        in_specs=[
            pl.BlockSpec((BTA, DIM), lambda i: (i, 0)),
            pl.BlockSpec((HID, DIM), lambda i: (0, 0)),
            pl.BlockSpec((HID, DIM), lambda i: (0, 0)),
            pl.BlockSpec((DIM, HID), lambda i: (0, 0)),
        ],
        out_specs=pl.BlockSpec((BTA, DIM), lambda i: (i, 0)),
        out_shape=jax.ShapeDtypeStruct((T, DIM), jnp.float32),
    )(x_flat, sw1, sw2, sw3)

    grid_spec = pltpu.PrefetchScalarGridSpec(
        num_scalar_prefetch=1,
        grid=(NB,),
        in_specs=[
            pl.BlockSpec((BT, DIM), lambda b, be: (b, 0)),
            pl.BlockSpec((1, HID, DIM), lambda b, be: (be[b], 0, 0)),
            pl.BlockSpec((1, HID, DIM), lambda b, be: (be[b], 0, 0)),
            pl.BlockSpec((1, DIM, HID), lambda b, be: (be[b], 0, 0)),
        ],
        out_specs=pl.BlockSpec((BT, DIM), lambda b, be: (b, 0)),
    )
    y_sorted = pl.pallas_call(
        _grouped_ffn_kernel,
        grid_spec=grid_spec,
        out_shape=jax.ShapeDtypeStruct((NP, DIM), jnp.float32),
    )(block_expert, x_sorted, w1b, w2b, w3b)

    g = jnp.take(y_sorted, jnp.concatenate([pos1, pos2]), axis=0)
    nc = T // BTC
    out = pl.pallas_call(
        _combine_kernel,
        grid=(nc,),
        in_specs=[
            pl.BlockSpec((BTC, DIM), lambda i: (i, 0)),
            pl.BlockSpec((BTC, DIM), lambda i: (i, 0)),
            pl.BlockSpec((BTC, DIM), lambda i: (i + nc, 0)),
            pl.BlockSpec((BTC, TOPK), lambda i: (i, 0)),
        ],
        out_specs=pl.BlockSpec((BTC, DIM), lambda i: (i, 0)),
        out_shape=jax.ShapeDtypeStruct((T, DIM), jnp.float32),
    )(shared_out, g, g, top_w)

    return (out.reshape(batch, seq, dim), logits)
